# baseline (speedup 1.0000x reference)
import functools

import jax
import jax.numpy as jnp
from jax import lax
from jax.experimental import pallas as pl
from jax.experimental.pallas import tpu as pltpu
from jax.experimental.pallas import tpu_sc as plsc

N = 10000
D_IN = 256
D_OUT = 40
HOP = 10

DPAD = 48
NVEC = DPAD // 16
NSUB = 16
NCORE = 2
NW = NSUB * NCORE
NPAD = 10240
ROWS_PT = NPAD // NSUB
CHUNK = 256
NCHUNK = 20
RB = 1000


def _front_body(mean_ref, w1t_ref, b1_ref, w2t_ref, ew2_ref, z_ref, c_ref):
    x = jnp.dot(mean_ref[...], w1t_ref[...],
                preferred_element_type=jnp.float32,
                precision=lax.Precision.HIGHEST)
    x = jnp.maximum(x + b1_ref[...], 0.0)
    z = jnp.dot(x, w2t_ref[...],
                preferred_element_type=jnp.float32,
                precision=lax.Precision.HIGHEST)
    z_ref[...] = z
    c_ref[...] = ew2_ref[...] * z


def _front(mean, w1t, b1, w2t, ew2):
    grid = (N // RB,)
    return pl.pallas_call(
        _front_body,
        grid=grid,
        in_specs=[
            pl.BlockSpec((RB, D_IN), lambda i: (i, 0)),
            pl.BlockSpec((D_IN, D_IN), lambda i: (0, 0)),
            pl.BlockSpec((1, D_IN), lambda i: (0, 0)),
            pl.BlockSpec((D_IN, DPAD), lambda i: (0, 0)),
            pl.BlockSpec((RB, 1), lambda i: (i, 0)),
        ],
        out_specs=[
            pl.BlockSpec((RB, DPAD), lambda i: (i, 0)),
            pl.BlockSpec((RB, DPAD), lambda i: (i, 0)),
        ],
        out_shape=[
            jax.ShapeDtypeStruct((N, DPAD), jnp.float32),
            jax.ShapeDtypeStruct((N, DPAD), jnp.float32),
        ],
    )(mean, w1t, b1, w2t, ew2)


def _softmax_body(g_ref, b2_ref, out_ref):
    g = g_ref[...] + b2_ref[...]
    m = jnp.max(g, axis=1, keepdims=True)
    e = jnp.exp(g - m)
    out_ref[...] = e / jnp.sum(e, axis=1, keepdims=True)


def _softmax(g, b2p):
    grid = (N // RB,)
    return pl.pallas_call(
        _softmax_body,
        grid=grid,
        in_specs=[
            pl.BlockSpec((RB, DPAD), lambda i: (i, 0)),
            pl.BlockSpec((1, DPAD), lambda i: (0, 0)),
        ],
        out_specs=pl.BlockSpec((RB, DPAD), lambda i: (i, 0)),
        out_shape=jax.ShapeDtypeStruct((N, DPAD), jnp.float32),
    )(g, b2p)


def _sc_body(zp, cp, cz, srcT, dstT, ewT, iotaT,
             g0, g1, p0, p1,
             src_v, dst_v, ew_v, iota_v,
             buf0, buf1, buf2, buf3,
             gn,
             gs0, gs1, gs2, gs3,
             ss0, ss1, ss2, ss3,
             xsem):
    cid = lax.axis_index("c")
    sid = lax.axis_index("s")
    wid = cid * NSUB + sid
    base = sid * ROWS_PT
    sl = pl.ds(base, ROWS_PT)
    bufs = (buf0, buf1, buf2, buf3)
    gsems = (gs0, gs1, gs2, gs3)
    ssems = (ss0, ss1, ss2, ss3)

    pltpu.sync_copy(srcT.at[wid], src_v)
    pltpu.sync_copy(dstT.at[wid], dst_v)
    pltpu.sync_copy(ewT.at[wid], ew_v)
    pltpu.sync_copy(iotaT.at[sid], iota_v)

    @pl.when(cid == 0)
    def _():
        pltpu.sync_copy(cp.at[sl], gn.at[sl])

    @pl.when(cid == 1)
    def _():
        pltpu.sync_copy(cz.at[sl], gn.at[sl])

    plsc.subcore_barrier()

    def _scatter_phase(gsrc):
        def g_issue(j, b):
            pltpu.async_copy(gsrc.at[src_v.at[j]], bufs[b], gsems[b])

        def g_drain(b):
            pltpu.make_async_copy(gsrc.at[src_v.at[0]], bufs[b], gsems[b]).wait()

        def s_issue(j, b):
            pltpu.async_copy(bufs[b], gn.at[dst_v.at[j]], ssems[b], add=True)

        def s_drain(b):
            pltpu.make_async_copy(bufs[b], gn.at[dst_v.at[0]], ssems[b]).wait()

        g_issue(0, 0)
        g_issue(1, 1)

        def quad(i, carry):
            for b in range(4):
                j = 4 * i + b
                jn = j + 2
                bn = (b + 2) % 4

                @pl.when(j >= 2)
                def _():
                    s_drain(bn)

                @pl.when(jn < NCHUNK)
                def _():
                    g_issue(jn, bn)

                g_drain(b)

                def gscale(g, c2):
                    gbase = g * 16
                    wv = ew_v[j, pl.ds(gbase, 16)]
                    for el in range(16):
                        w = wv[el]
                        for k in range(NVEC):
                            sl16 = pl.ds(k * 16, 16)
                            bufs[b][gbase + el, sl16] = bufs[b][gbase + el, sl16] * w
                    return c2
                lax.fori_loop(0, CHUNK // 16, gscale, 0)

                s_issue(j, b)
            return carry
        lax.fori_loop(0, NCHUNK // 4, quad, 0)
        s_drain((NCHUNK - 2) % 4)
        s_drain((NCHUNK - 1) % 4)

    def _combine(p_other):
        for q in range(ROWS_PT // 128):
            qsl = pl.ds(base + 128 * q, 128)
            bsl = pl.ds(0, 128)
            pltpu.sync_copy(p_other.at[qsl], buf0.at[bsl])
            pltpu.sync_copy(buf0.at[bsl], gn.at[iota_v.at[q]], add=True)

    def do_hop(srcref0, srcref1):
        @pl.when(cid == 0)
        def _():
            _scatter_phase(srcref0)

        @pl.when(cid == 1)
        def _():
            _scatter_phase(srcref1)

        plsc.subcore_barrier()

        @pl.when(cid == 0)
        def _():
            pltpu.sync_copy(gn.at[sl], p0.at[sl])

        @pl.when(cid == 1)
        def _():
            pltpu.sync_copy(gn.at[sl], p1.at[sl])

        pl.semaphore_signal(xsem, 1, device_id={"c": 1 - cid, "s": sid})
        pl.semaphore_wait(xsem, 1)

        @pl.when(cid == 0)
        def _():
            _combine(p1)
            pltpu.sync_copy(gn.at[sl], g0.at[sl])
            pltpu.sync_copy(cp.at[sl], gn.at[sl])

        @pl.when(cid == 1)
        def _():
            _combine(p0)
            pltpu.sync_copy(gn.at[sl], g1.at[sl])
            pltpu.sync_copy(cz.at[sl], gn.at[sl])

        plsc.subcore_barrier()

    do_hop(zp, zp)

    def hop(h, carry):
        do_hop(g0, g1)
        return carry

    lax.fori_loop(0, HOP - 1, hop, 0)


def _sc_hops(zp, cp, cz, srcT, dstT, ewT, iotaT):
    mesh = plsc.VectorSubcoreMesh(core_axis_name="c", subcore_axis_name="s")
    return pl.kernel(
        _sc_body,
        out_type=[
            jax.ShapeDtypeStruct((NPAD, DPAD), jnp.float32),
            jax.ShapeDtypeStruct((NPAD, DPAD), jnp.float32),
            jax.ShapeDtypeStruct((NPAD, DPAD), jnp.float32),
            jax.ShapeDtypeStruct((NPAD, DPAD), jnp.float32),
        ],
        mesh=mesh,
        compiler_params=pltpu.CompilerParams(
            use_tc_tiling_on_sc=False, needs_layout_passes=False),
        scratch_types=[
            pltpu.VMEM((NCHUNK, CHUNK), jnp.int32),
            pltpu.VMEM((NCHUNK, CHUNK), jnp.int32),
            pltpu.VMEM((NCHUNK, CHUNK), jnp.float32),
            pltpu.VMEM((ROWS_PT // 128, 128), jnp.int32),
            pltpu.VMEM((CHUNK, DPAD), jnp.float32),
            pltpu.VMEM((CHUNK, DPAD), jnp.float32),
            pltpu.VMEM((CHUNK, DPAD), jnp.float32),
            pltpu.VMEM((CHUNK, DPAD), jnp.float32),
            pltpu.VMEM_SHARED((NPAD, DPAD), jnp.float32),
        ] + [pltpu.SemaphoreType.DMA] * 8
          + [pltpu.SemaphoreType.REGULAR],
    )(zp, cp, cz, srcT, dstT, ewT, iotaT)


def kernel(mean, edge_index, edge_weight1, edge_weight2, W1, b1, W2, b2):
    w1t = W1.T
    w2t = jnp.zeros((D_IN, DPAD), jnp.float32).at[:, :D_OUT].set(W2.T)
    b1r = b1.reshape(1, D_IN)
    b2p = jnp.concatenate([b2, jnp.full((DPAD - D_OUT,), -1e30, jnp.float32)])
    b2p = b2p.reshape(1, DPAD)

    z, c = _front(mean, w1t, b1r, w2t, edge_weight2)

    zp = jnp.pad(z, ((0, NPAD - N), (0, 0)))
    cp = jnp.pad(c, ((0, NPAD - N), (0, 0)))
    cz = jnp.zeros((NPAD, DPAD), jnp.float32)

    epad = NW * NCHUNK * CHUNK - edge_index.shape[1]
    src = jnp.concatenate(
        [edge_index[0], jnp.zeros((epad,), jnp.int32)]).reshape(NW, NCHUNK, CHUNK)
    dst = jnp.concatenate(
        [edge_index[1], jnp.full((epad,), N, jnp.int32)]).reshape(NW, NCHUNK, CHUNK)
    ew = jnp.concatenate(
        [edge_weight1, jnp.zeros((epad,), jnp.float32)]).reshape(NW, NCHUNK, CHUNK)
    iota = jnp.arange(NPAD, dtype=jnp.int32).reshape(NSUB, ROWS_PT // 128, 128)

    g0, _, _, _ = _sc_hops(zp, cp, cz, src, dst, ew, iota)

    y = _softmax(g0[:N], b2p)
    return y[:, :D_OUT]


# speedup vs baseline: 1.4576x; 1.4576x over previous
import functools

import jax
import jax.numpy as jnp
from jax import lax
from jax.experimental import pallas as pl
from jax.experimental.pallas import tpu as pltpu
from jax.experimental.pallas import tpu_sc as plsc

N = 10000
D_IN = 256
D_OUT = 40
HOP = 10

DPAD = 48
DA = 32
DB = 16
NSUB = 16
NPAD = 10112
ROWS_PT = NPAD // NSUB
CHUNK = 256
NCHUNK = 40
RB = 1000


def _front_body(mean_ref, w1t_ref, b1_ref, w2t_ref, ew2_ref, z_ref, c_ref):
    x = jnp.dot(mean_ref[...], w1t_ref[...],
                preferred_element_type=jnp.float32,
                precision=lax.Precision.HIGHEST)
    x = jnp.maximum(x + b1_ref[...], 0.0)
    z = jnp.dot(x, w2t_ref[...],
                preferred_element_type=jnp.float32,
                precision=lax.Precision.HIGHEST)
    z_ref[...] = z
    c_ref[...] = ew2_ref[...] * z


def _front(mean, w1t, b1, w2t, ew2):
    grid = (N // RB,)
    return pl.pallas_call(
        _front_body,
        grid=grid,
        in_specs=[
            pl.BlockSpec((RB, D_IN), lambda i: (i, 0)),
            pl.BlockSpec((D_IN, D_IN), lambda i: (0, 0)),
            pl.BlockSpec((1, D_IN), lambda i: (0, 0)),
            pl.BlockSpec((D_IN, DPAD), lambda i: (0, 0)),
            pl.BlockSpec((RB, 1), lambda i: (i, 0)),
        ],
        out_specs=[
            pl.BlockSpec((RB, DPAD), lambda i: (i, 0)),
            pl.BlockSpec((RB, DPAD), lambda i: (i, 0)),
        ],
        out_shape=[
            jax.ShapeDtypeStruct((N, DPAD), jnp.float32),
            jax.ShapeDtypeStruct((N, DPAD), jnp.float32),
        ],
    )(mean, w1t, b1, w2t, ew2)


def _softmax_body(ga_ref, gb_ref, b2_ref, out_ref):
    g = jnp.concatenate([ga_ref[...], gb_ref[...]], axis=1) + b2_ref[...]
    m = jnp.max(g, axis=1, keepdims=True)
    e = jnp.exp(g - m)
    out_ref[...] = e / jnp.sum(e, axis=1, keepdims=True)


def _softmax(ga, gb, b2p):
    grid = (N // RB,)
    return pl.pallas_call(
        _softmax_body,
        grid=grid,
        in_specs=[
            pl.BlockSpec((RB, DA), lambda i: (i, 0)),
            pl.BlockSpec((RB, DB), lambda i: (i, 0)),
            pl.BlockSpec((1, DPAD), lambda i: (0, 0)),
        ],
        out_specs=pl.BlockSpec((RB, DPAD), lambda i: (i, 0)),
        out_shape=jax.ShapeDtypeStruct((N, DPAD), jnp.float32),
    )(ga, gb, b2p)


def _sc_body(zA, zB, cA, cB, srcT, dstT, ewT,
             gA, gB,
             src_v, dst_v, ew_v,
             rowsA0, rowsA1, rowsA2, rowsA3,
             rowsB0, rowsB1, rowsB2, rowsB3,
             gnA, gnB,
             gs0, gs1, gs2, gs3,
             gt0, gt1, gt2, gt3,
             ss0, ss1, ss2, ss3):
    cid = lax.axis_index("c")
    sid = lax.axis_index("s")
    base = sid * ROWS_PT
    sl = pl.ds(base, ROWS_PT)

    pltpu.sync_copy(srcT.at[sid], src_v)
    pltpu.sync_copy(dstT.at[sid], dst_v)
    pltpu.sync_copy(ewT.at[sid], ew_v)

    @pl.when(cid == 0)
    def _():
        pltpu.sync_copy(cA.at[sl], gnA.at[sl])

    @pl.when(cid == 1)
    def _():
        pltpu.sync_copy(cB.at[sl], gnB.at[sl])

    plsc.subcore_barrier()

    gsems = (gs0, gs1, gs2, gs3)
    gsems2 = (gt0, gt1, gt2, gt3)
    ssems = (ss0, ss1, ss2, ss3)
    H = CHUNK // 2
    lo = pl.ds(0, H)
    hi = pl.ds(H, H)

    def _scatter_phase(gsrc, gn, bufs, nvec):
        def g_issue(j, b):
            pltpu.async_copy(gsrc.at[src_v.at[j, lo]], bufs[b].at[lo], gsems[b])
            pltpu.async_copy(gsrc.at[src_v.at[j, hi]], bufs[b].at[hi], gsems2[b])

        def g_drain(b):
            pltpu.make_async_copy(gsrc.at[src_v.at[0, lo]], bufs[b].at[lo], gsems[b]).wait()
            pltpu.make_async_copy(gsrc.at[src_v.at[0, hi]], bufs[b].at[hi], gsems2[b]).wait()

        def s_issue(j, b):
            pltpu.async_copy(bufs[b], gn.at[dst_v.at[j]], ssems[b], add=True)

        def s_drain(b):
            pltpu.make_async_copy(bufs[b], gn.at[dst_v.at[0]], ssems[b]).wait()

        g_issue(0, 0)
        g_issue(1, 1)

        def quad(i, carry):
            for b in range(4):
                j = 4 * i + b
                jn = j + 2
                bn = (b + 2) % 4

                @pl.when(j >= 2)
                def _():
                    s_drain(bn)

                @pl.when(jn < NCHUNK)
                def _():
                    g_issue(jn, bn)

                g_drain(b)

                def gscale(g, c2):
                    base = g * 16
                    wv = ew_v[j, pl.ds(base, 16)]
                    for el in range(16):
                        w = wv[el]
                        for k in range(nvec):
                            sl16 = pl.ds(k * 16, 16)
                            bufs[b][base + el, sl16] = bufs[b][base + el, sl16] * w
                    return c2
                lax.fori_loop(0, CHUNK // 16, gscale, 0)

                s_issue(j, b)
            return carry
        lax.fori_loop(0, NCHUNK // 4, quad, 0)
        s_drain((NCHUNK - 2) % 4)
        s_drain((NCHUNK - 1) % 4)

    def _copyout(c_hbm, gout, gn):
        pltpu.sync_copy(gn.at[sl], gout.at[sl])
        pltpu.sync_copy(c_hbm.at[sl], gn.at[sl])

    def do_hop(srcA, srcB):
        @pl.when(cid == 0)
        def _():
            _scatter_phase(srcA, gnA, (rowsA0, rowsA1, rowsA2, rowsA3), 2)

        @pl.when(cid == 1)
        def _():
            _scatter_phase(srcB, gnB, (rowsB0, rowsB1, rowsB2, rowsB3), 1)

        plsc.subcore_barrier()

        @pl.when(cid == 0)
        def _():
            _copyout(cA, gA, gnA)

        @pl.when(cid == 1)
        def _():
            _copyout(cB, gB, gnB)

        plsc.subcore_barrier()

    do_hop(zA, zB)

    def hop(h, carry):
        do_hop(gA, gB)
        return carry

    lax.fori_loop(0, HOP - 1, hop, 0)


def _sc_hops(zA, zB, cA, cB, srcT, dstT, ewT):
    mesh = plsc.VectorSubcoreMesh(core_axis_name="c", subcore_axis_name="s")
    return pl.kernel(
        _sc_body,
        out_type=[
            jax.ShapeDtypeStruct((NPAD, DA), jnp.float32),
            jax.ShapeDtypeStruct((NPAD, DB), jnp.float32),
        ],
        mesh=mesh,
        compiler_params=pltpu.CompilerParams(
            use_tc_tiling_on_sc=False, needs_layout_passes=False),
        scratch_types=[
            pltpu.VMEM((NCHUNK, CHUNK), jnp.int32),
            pltpu.VMEM((NCHUNK, CHUNK), jnp.int32),
            pltpu.VMEM((NCHUNK, CHUNK), jnp.float32),
            pltpu.VMEM((CHUNK, DA), jnp.float32),
            pltpu.VMEM((CHUNK, DA), jnp.float32),
            pltpu.VMEM((CHUNK, DA), jnp.float32),
            pltpu.VMEM((CHUNK, DA), jnp.float32),
            pltpu.VMEM((CHUNK, DB), jnp.float32),
            pltpu.VMEM((CHUNK, DB), jnp.float32),
            pltpu.VMEM((CHUNK, DB), jnp.float32),
            pltpu.VMEM((CHUNK, DB), jnp.float32),
            pltpu.VMEM_SHARED((NPAD, DA), jnp.float32),
            pltpu.VMEM_SHARED((NPAD, DB), jnp.float32),
        ] + [pltpu.SemaphoreType.DMA] * 12,
    )(zA, zB, cA, cB, srcT, dstT, ewT)


def kernel(mean, edge_index, edge_weight1, edge_weight2, W1, b1, W2, b2):
    w1t = W1.T
    w2t = jnp.zeros((D_IN, DPAD), jnp.float32).at[:, :D_OUT].set(W2.T)
    b1r = b1.reshape(1, D_IN)
    b2p = jnp.concatenate([b2, jnp.full((DPAD - D_OUT,), -1e30, jnp.float32)])
    b2p = b2p.reshape(1, DPAD)

    z, c = _front(mean, w1t, b1r, w2t, edge_weight2)

    zp = jnp.pad(z, ((0, NPAD - N), (0, 0)))
    cp = jnp.pad(c, ((0, NPAD - N), (0, 0)))
    zA, zB = zp[:, :DA], zp[:, DA:]
    cA, cB = cp[:, :DA], cp[:, DA:]

    epad = NSUB * NCHUNK * CHUNK - edge_index.shape[1]
    src = jnp.concatenate(
        [edge_index[0], jnp.zeros((epad,), jnp.int32)]).reshape(NSUB, NCHUNK, CHUNK)
    dst = jnp.concatenate(
        [edge_index[1], jnp.full((epad,), N, jnp.int32)]).reshape(NSUB, NCHUNK, CHUNK)
    ew = jnp.concatenate(
        [edge_weight1, jnp.zeros((epad,), jnp.float32)]).reshape(NSUB, NCHUNK, CHUNK)

    gA, gB = _sc_hops(zA, zB, cA, cB, src, dst, ew)

    y = _softmax(gA[:N], gB[:N], b2p)
    return y[:, :D_OUT]
